# baseline (speedup 1.0000x reference)
import functools

import jax
import jax.numpy as jnp
from jax import lax
from jax.experimental import pallas as pl
from jax.experimental.pallas import tpu as pltpu
from jax.experimental.pallas import tpu_sc as plsc

N = 10000
E = 160000
D = 256
DH = 128
EPS = 1e-5

NCORES = 2
NSUB = 16
NTILES = NCORES * NSUB

DEG_EPT = E // NTILES
PROP_EPT = E // NSUB
CH = 80
NCHUNK = PROP_EPT // CH
ZCH = CH

_SC_MESH = plsc.VectorSubcoreMesh(core_axis_name="c", subcore_axis_name="s")
_SC_PARAMS = pltpu.CompilerParams(needs_layout_passes=False)


def _zero16():
    return jnp.zeros((16,), jnp.float32)


@functools.partial(
    pl.kernel,
    mesh=_SC_MESH,
    compiler_params=_SC_PARAMS,
    out_type=jax.ShapeDtypeStruct((NTILES * N,), jnp.float32),
    scratch_types=[
        pltpu.VMEM((DEG_EPT + 16,), jnp.int32),
        pltpu.VMEM((DEG_EPT + 16,), jnp.float32),
        pltpu.VMEM((N,), jnp.float32),
    ],
)
def _deg_kernel(col_hbm, w_hbm, degp_hbm, colb, wb, degl):
    c = lax.axis_index("c")
    s = lax.axis_index("s")
    wid = s * NCORES + c
    base = wid * DEG_EPT

    pltpu.sync_copy(col_hbm.at[pl.ds(base, DEG_EPT)], colb.at[pl.ds(0, DEG_EPT)])
    pltpu.sync_copy(w_hbm.at[pl.ds(base, DEG_EPT)], wb.at[pl.ds(0, DEG_EPT)])

    def zbody(i, carry):
        degl[pl.ds(i * 16, 16)] = _zero16()
        return carry

    lax.fori_loop(0, N // 16, zbody, 0)

    nfull = DEG_EPT // 16
    rem = DEG_EPT - nfull * 16

    def sbody(j, carry):
        idx = colb[pl.ds(j * 16, 16)]
        vals = wb[pl.ds(j * 16, 16)]
        plsc.addupdate_scatter(degl, [idx], vals)
        return carry

    lax.fori_loop(0, nfull, sbody, 0)

    lane = jax.lax.broadcasted_iota(jnp.int32, (16,), 0)
    mask = lane < rem
    idx = colb[pl.ds(nfull * 16, 16)]
    idx = jnp.minimum(jnp.maximum(idx, 0), N - 1)
    vals = jnp.where(mask, wb[pl.ds(nfull * 16, 16)], 0.0)
    plsc.addupdate_scatter(degl, [idx], vals, mask=mask)

    pltpu.sync_copy(degl, degp_hbm.at[pl.ds(wid * N, N)])


@functools.partial(
    pl.kernel,
    mesh=_SC_MESH,
    compiler_params=_SC_PARAMS,
    out_type=jax.ShapeDtypeStruct((NCORES * N, DH), jnp.float32),
    scratch_types=[
        pltpu.VMEM_SHARED((N, DH), jnp.float32),
        pltpu.VMEM((NCHUNK, CH), jnp.int32),
        pltpu.VMEM((2, 2, CH), jnp.int32),
        pltpu.VMEM((CH, DH), jnp.float32),
        pltpu.VMEM((CH, DH), jnp.float32),
        pltpu.SemaphoreType.DMA,
        pltpu.SemaphoreType.DMA,
        pltpu.SemaphoreType.DMA,
        pltpu.SemaphoreType.DMA,
        pltpu.SemaphoreType.DMA,
        pltpu.SemaphoreType.DMA,
    ],
)
def _prop_kernel(hsf_hbm, row_hbm, cw_hbm, out_hbm, acc_sh,
                 rowb, cwsm, rows0, rows1,
                 sg0, sg1, scw0, scw1, ss0, ss1):
    c = lax.axis_index("c")
    s = lax.axis_index("s")
    cN = c * N

    pltpu.sync_copy(row_hbm.at[s], rowb)

    def obody(i, carry):
        for v in range(CH // 16):
            rowb[i, pl.ds(v * 16, 16)] = rowb[i, pl.ds(v * 16, 16)] + cN
        return carry

    lax.fori_loop(0, NCHUNK, obody, 0)

    def zbody(i, carry):
        for f in range(DH // 16):
            rows0[i, pl.ds(f * 16, 16)] = _zero16()
        return carry

    lax.fori_loop(0, ZCH, zbody, 0)
    nrowchunks = N // ZCH
    for k in range((nrowchunks + NSUB - 1) // NSUB):
        j = s + k * NSUB

        @pl.when(j < nrowchunks)
        def _():
            pltpu.sync_copy(rows0, acc_sh.at[pl.ds(j * ZCH, ZCH)])

    plsc.subcore_barrier()

    def start(k, bufs):
        rbuf, semg, cwb, _cidx, semcw, sems = bufs
        pltpu.async_copy(hsf_hbm.at[rowb.at[k]], rbuf, semg)
        pltpu.async_copy(cw_hbm.at[s, k], cwb, semcw)

    def process(k, bufs, nbufs):
        rbuf, semg, cwb, cidx, semcw, sems = bufs

        @pl.when(k >= 1)
        def _():
            pltpu.make_async_copy(
                nbufs[0], acc_sh.at[pl.ds(0, CH)], nbufs[5]).wait()

        @pl.when(k + 1 < NCHUNK)
        def _():
            start(k + 1, nbufs)

        pltpu.make_async_copy(hsf_hbm.at[pl.ds(0, CH)], rbuf, semg).wait()
        pltpu.make_async_copy(cw_hbm.at[0, 0], cwb, semcw).wait()

        b16 = jnp.full((16,), 1, jnp.int32)

        @plsc.parallel_loop(0, CH, 1, unroll=4)
        def scale(j):
            wi = plsc.load_gather(cwb, [b16, jnp.full((16,), j, jnp.int32)])
            wv = plsc.bitcast(wi, jnp.float32)
            for f in range(DH // 16):
                rbuf[j, pl.ds(f * 16, 16)] = rbuf[j, pl.ds(f * 16, 16)] * wv

        pltpu.async_copy(rbuf, acc_sh.at[cidx], sems, add=True)

    bufs0 = (rows0, sg0, cwsm.at[0], cwsm.at[0, 0], scw0, ss0)
    bufs1 = (rows1, sg1, cwsm.at[1], cwsm.at[1, 0], scw1, ss1)
    start(0, bufs0)

    def gbody(g, carry):
        process(2 * g, bufs0, bufs1)
        process(2 * g + 1, bufs1, bufs0)
        return carry

    lax.fori_loop(0, (NCHUNK - 1) // 2, gbody, 0)
    process(NCHUNK - 1, bufs0, bufs1)
    pltpu.make_async_copy(rows0, acc_sh.at[pl.ds(0, CH)], ss0).wait()

    plsc.subcore_barrier()

    for k in range((nrowchunks + NSUB - 1) // NSUB):
        j = s + k * NSUB

        @pl.when(j < nrowchunks)
        def _():
            pltpu.sync_copy(
                acc_sh.at[pl.ds(j * ZCH, ZCH)],
                out_hbm.at[pl.ds(c * N + j * ZCH, ZCH)],
            )


BM = 512
GRID = (N + BM - 1) // BM

_full = lambda i: (0, 0)


def _dinv_from(degp):
    return lax.rsqrt(1.0 + jnp.sum(degp, axis=0))[:, None]


def _tc1_body(x_ref, w1_ref, degp_ref, hs_ref):
    dinv = _dinv_from(degp_ref[...])
    r = jnp.dot(x_ref[...], w1_ref[...], preferred_element_type=jnp.float32,
                precision=lax.Precision.DEFAULT)
    r = r * dinv
    hs_ref[0] = r[:, :DH]
    hs_ref[1] = r[:, DH:]


def _ln_relu(t, g, b):
    mu = jnp.mean(t, axis=1, keepdims=True)
    var = jnp.mean((t - mu) ** 2, axis=1, keepdims=True)
    return jnp.maximum((t - mu) * lax.rsqrt(var + EPS) * g + b, 0.0)


def _tc2_body(acc_ref, hs_ref, degp_ref, w2_ref, b1_ref, g1_ref, bb1_ref,
              h1_ref, hs2_ref):
    dinv = _dinv_from(degp_ref[...])
    acc = jnp.concatenate([acc_ref[0], acc_ref[1]], axis=1)
    hs = jnp.concatenate([hs_ref[0], hs_ref[1]], axis=1)
    t = dinv * (acc + hs) + b1_ref[...]
    h1 = _ln_relu(t, g1_ref[...], bb1_ref[...])
    h1_ref[...] = h1
    r = jnp.dot(h1, w2_ref[...], preferred_element_type=jnp.float32,
                precision=lax.Precision.DEFAULT)
    r = r * dinv
    hs2_ref[0] = r[:, :DH]
    hs2_ref[1] = r[:, DH:]


def _tc3a_body(x_ref, h1_ref, fc1ab_ref, fc1b_ref, u0_ref):
    fc1ab = fc1ab_ref[...]
    dot = functools.partial(jnp.dot, preferred_element_type=jnp.float32,
                            precision=lax.Precision.DEFAULT)
    u0_ref[...] = (dot(x_ref[...], fc1ab[:D]) + dot(h1_ref[...], fc1ab[D:])
                   + fc1b_ref[...])


def _tc3b_body(u0_ref, acc_ref, hs_ref, degp_ref, b2_ref, g2_ref,
               bb2_ref, fc1c_ref, fc2w_ref, fc2b_ref, out_ref):
    dinv = _dinv_from(degp_ref[...])
    acc = jnp.concatenate([acc_ref[0], acc_ref[1]], axis=1)
    hs = jnp.concatenate([hs_ref[0], hs_ref[1]], axis=1)
    t = dinv * (acc + hs) + b2_ref[...]
    h2 = _ln_relu(t, g2_ref[...], bb2_ref[...])
    dot = functools.partial(jnp.dot, preferred_element_type=jnp.float32,
                            precision=lax.Precision.DEFAULT)
    u = jnp.maximum(u0_ref[...] + dot(h2, fc1c_ref[...]), 0.0)
    out_ref[...] = dot(u, fc2w_ref[...]) + fc2b_ref[...]


def _row_spec(width):
    return pl.BlockSpec((BM, width), lambda i: (i, 0))


_half_spec = pl.BlockSpec((2, BM, DH), lambda i: (0, i, 0))
_degp_spec = pl.BlockSpec((NTILES, BM), lambda i: (0, i))


def _vec_spec(width):
    return pl.BlockSpec((1, width), lambda i: (0, 0))


def _mat_spec(h, w):
    return pl.BlockSpec((h, w), _full)


def _tc1_call(x, W1, degp):
    return pl.pallas_call(
        _tc1_body,
        grid=(GRID,),
        in_specs=[_row_spec(D), _mat_spec(D, D), _degp_spec],
        out_specs=_half_spec,
        out_shape=jax.ShapeDtypeStruct((2, N, DH), jnp.float32),
    )(x, W1, degp)


def _tc2_call(acc1, hs1, degp, W2, b1, g1, bb1):
    return pl.pallas_call(
        _tc2_body,
        grid=(GRID,),
        in_specs=[_half_spec, _half_spec, _degp_spec, _mat_spec(D, D),
                  _vec_spec(D), _vec_spec(D), _vec_spec(D)],
        out_specs=[_row_spec(D), _half_spec],
        out_shape=[
            jax.ShapeDtypeStruct((N, D), jnp.float32),
            jax.ShapeDtypeStruct((2, N, DH), jnp.float32),
        ],
    )(acc1, hs1, degp, W2, b1, g1, bb1)


def _tc3a_call(x, h1, fc1_W, fc1_b):
    return pl.pallas_call(
        _tc3a_body,
        grid=(GRID,),
        in_specs=[_row_spec(D), _row_spec(D), _mat_spec(2 * D, D),
                  _vec_spec(D)],
        out_specs=_row_spec(D),
        out_shape=jax.ShapeDtypeStruct((N, D), jnp.float32),
    )(x, h1, fc1_W[:2 * D], fc1_b)


def _tc3b_call(u0, acc2, hs2, degp, b2, g2, bb2, fc1_W, fc2_W, fc2_b):
    return pl.pallas_call(
        _tc3b_body,
        grid=(GRID,),
        in_specs=[_row_spec(D), _half_spec, _half_spec,
                  _degp_spec, _vec_spec(D), _vec_spec(D), _vec_spec(D),
                  _mat_spec(D, D), _mat_spec(D, D), _vec_spec(D)],
        out_specs=_row_spec(D),
        out_shape=jax.ShapeDtypeStruct((N, D), jnp.float32),
    )(u0, acc2, hs2, degp, b2, g2, bb2, fc1_W[2 * D:], fc2_W, fc2_b)


def kernel(x, adj, weight, W1, b1, ln1_g, ln1_b, W2, b2, ln2_g, ln2_b,
           fc1_W, fc1_b, fc2_W, fc2_b):
    row = adj[0].astype(jnp.int32)
    col = adj[1].astype(jnp.int32)
    w = weight.astype(jnp.float32)

    b1r = b1.reshape(1, D)
    g1r = ln1_g.reshape(1, D)
    bb1r = ln1_b.reshape(1, D)
    b2r = b2.reshape(1, D)
    g2r = ln2_g.reshape(1, D)
    bb2r = ln2_b.reshape(1, D)
    fc1br = fc1_b.reshape(1, D)
    fc2br = fc2_b.reshape(1, D)

    degp = _deg_kernel(col, w).reshape(NTILES, N)
    row3 = row.reshape(NSUB, NCHUNK, CH)
    cw = jnp.stack([col.reshape(NSUB, NCHUNK, CH),
                    lax.bitcast_convert_type(w, jnp.int32)
                    .reshape(NSUB, NCHUNK, CH)], axis=2)
    hs1 = _tc1_call(x, W1, degp)
    acc1 = _prop_kernel(hs1.reshape(2 * N, DH), row3, cw)
    acc1 = acc1.reshape(2, N, DH)
    h1, hs2 = _tc2_call(acc1, hs1, degp, W2, b1r, g1r, bb1r)
    acc2 = _prop_kernel(hs2.reshape(2 * N, DH), row3, cw)
    acc2 = acc2.reshape(2, N, DH)
    u0 = _tc3a_call(x, h1, fc1_W, fc1br)
    return _tc3b_call(u0, acc2, hs2, degp, b2r, g2r, bb2r,
                      fc1_W, fc2_W, fc2br)


# speedup vs baseline: 1.0480x; 1.0480x over previous
import functools

import jax
import jax.numpy as jnp
from jax import lax
from jax.experimental import pallas as pl
from jax.experimental.pallas import tpu as pltpu
from jax.experimental.pallas import tpu_sc as plsc

N = 10000
E = 160000
D = 256
DH = 128
EPS = 1e-5

NCORES = 2
NSUB = 16
NTILES = NCORES * NSUB

DEG_EPT = E // NTILES
PROP_EPT = E // NSUB
CH = 80
NCHUNK = PROP_EPT // CH
ZCH = CH

_SC_MESH = plsc.VectorSubcoreMesh(core_axis_name="c", subcore_axis_name="s")
_SC_PARAMS = pltpu.CompilerParams(needs_layout_passes=False)


def _zero16():
    return jnp.zeros((16,), jnp.float32)


@functools.partial(
    pl.kernel,
    mesh=_SC_MESH,
    compiler_params=_SC_PARAMS,
    out_type=jax.ShapeDtypeStruct((NTILES * N,), jnp.float32),
    scratch_types=[
        pltpu.VMEM((DEG_EPT + 16,), jnp.int32),
        pltpu.VMEM((DEG_EPT + 16,), jnp.float32),
        pltpu.VMEM((N,), jnp.float32),
    ],
)
def _deg_kernel(col_hbm, w_hbm, degp_hbm, colb, wb, degl):
    c = lax.axis_index("c")
    s = lax.axis_index("s")
    wid = s * NCORES + c
    base = wid * DEG_EPT

    pltpu.sync_copy(col_hbm.at[pl.ds(base, DEG_EPT)], colb.at[pl.ds(0, DEG_EPT)])
    pltpu.sync_copy(w_hbm.at[pl.ds(base, DEG_EPT)], wb.at[pl.ds(0, DEG_EPT)])

    def zbody(i, carry):
        degl[pl.ds(i * 16, 16)] = _zero16()
        return carry

    lax.fori_loop(0, N // 16, zbody, 0)

    nfull = DEG_EPT // 16
    rem = DEG_EPT - nfull * 16

    def sbody(j, carry):
        idx = colb[pl.ds(j * 16, 16)]
        vals = wb[pl.ds(j * 16, 16)]
        plsc.addupdate_scatter(degl, [idx], vals)
        return carry

    lax.fori_loop(0, nfull, sbody, 0)

    lane = jax.lax.broadcasted_iota(jnp.int32, (16,), 0)
    mask = lane < rem
    idx = colb[pl.ds(nfull * 16, 16)]
    idx = jnp.minimum(jnp.maximum(idx, 0), N - 1)
    vals = jnp.where(mask, wb[pl.ds(nfull * 16, 16)], 0.0)
    plsc.addupdate_scatter(degl, [idx], vals, mask=mask)

    pltpu.sync_copy(degl, degp_hbm.at[pl.ds(wid * N, N)])


@functools.partial(
    pl.kernel,
    mesh=_SC_MESH,
    compiler_params=_SC_PARAMS,
    out_type=jax.ShapeDtypeStruct((NCORES * N, DH), jnp.float32),
    scratch_types=[
        pltpu.VMEM_SHARED((N, DH), jnp.float32),
        pltpu.VMEM((NCHUNK, CH), jnp.int32),
        pltpu.VMEM((2, CH), jnp.float32),
        pltpu.VMEM((2, CH), jnp.int32),
        pltpu.VMEM((CH, DH), jnp.float32),
        pltpu.VMEM((CH, DH), jnp.float32),
        pltpu.SemaphoreType.DMA,
        pltpu.SemaphoreType.DMA,
        pltpu.SemaphoreType.DMA,
        pltpu.SemaphoreType.DMA,
        pltpu.SemaphoreType.DMA,
        pltpu.SemaphoreType.DMA,
        pltpu.SemaphoreType.DMA,
        pltpu.SemaphoreType.DMA,
    ],
)
def _prop_kernel(hsf_hbm, row_hbm, col_hbm, w_hbm, out_hbm, acc_sh,
                 rowb, wsm, colsm, rows0, rows1,
                 sg0, sg1, sc0, sc1, sw0, sw1, ss0, ss1):
    c = lax.axis_index("c")
    s = lax.axis_index("s")
    cN = c * N

    pltpu.sync_copy(row_hbm.at[s], rowb)

    def obody(i, carry):
        for v in range(CH // 16):
            rowb[i, pl.ds(v * 16, 16)] = rowb[i, pl.ds(v * 16, 16)] + cN
        return carry

    lax.fori_loop(0, NCHUNK, obody, 0)

    def zbody(i, carry):
        for f in range(DH // 16):
            rows0[i, pl.ds(f * 16, 16)] = _zero16()
        return carry

    lax.fori_loop(0, ZCH, zbody, 0)
    nrowchunks = N // ZCH
    for k in range((nrowchunks + NSUB - 1) // NSUB):
        j = s + k * NSUB

        @pl.when(j < nrowchunks)
        def _():
            pltpu.sync_copy(rows0, acc_sh.at[pl.ds(j * ZCH, ZCH)])

    plsc.subcore_barrier()

    def start(k, bufs):
        rbuf, semg, cbuf, semc, wbuf, semw, sems = bufs
        ebase = s * PROP_EPT + k * CH
        pltpu.async_copy(hsf_hbm.at[rowb.at[k]], rbuf, semg)
        pltpu.async_copy(col_hbm.at[pl.ds(ebase, CH)], cbuf, semc)
        pltpu.async_copy(w_hbm.at[pl.ds(ebase, CH)], wbuf, semw)

    def process(k, bufs, nbufs):
        rbuf, semg, cbuf, semc, wbuf, semw, sems = bufs

        @pl.when(k >= 1)
        def _():
            pltpu.make_async_copy(
                nbufs[0], acc_sh.at[pl.ds(0, CH)], nbufs[6]).wait()

        @pl.when(k + 1 < NCHUNK)
        def _():
            start(k + 1, nbufs)

        pltpu.make_async_copy(hsf_hbm.at[pl.ds(0, CH)], rbuf, semg).wait()
        pltpu.make_async_copy(w_hbm.at[pl.ds(0, CH)], wbuf, semw).wait()

        @plsc.parallel_loop(0, CH, 1, unroll=4)
        def scale(j):
            wv = plsc.load_gather(wbuf, [jnp.full((16,), j, jnp.int32)])
            for f in range(DH // 16):
                rbuf[j, pl.ds(f * 16, 16)] = rbuf[j, pl.ds(f * 16, 16)] * wv

        pltpu.make_async_copy(col_hbm.at[pl.ds(0, CH)], cbuf, semc).wait()
        pltpu.async_copy(rbuf, acc_sh.at[cbuf], sems, add=True)

    bufs0 = (rows0, sg0, colsm.at[0], sc0, wsm.at[0], sw0, ss0)
    bufs1 = (rows1, sg1, colsm.at[1], sc1, wsm.at[1], sw1, ss1)
    start(0, bufs0)

    def gbody(g, carry):
        process(2 * g, bufs0, bufs1)
        process(2 * g + 1, bufs1, bufs0)
        return carry

    lax.fori_loop(0, (NCHUNK - 1) // 2, gbody, 0)
    process(NCHUNK - 1, bufs0, bufs1)
    pltpu.make_async_copy(rows0, acc_sh.at[pl.ds(0, CH)], ss0).wait()

    plsc.subcore_barrier()

    for k in range((nrowchunks + NSUB - 1) // NSUB):
        j = s + k * NSUB

        @pl.when(j < nrowchunks)
        def _():
            pltpu.sync_copy(
                acc_sh.at[pl.ds(j * ZCH, ZCH)],
                out_hbm.at[pl.ds(c * N + j * ZCH, ZCH)],
            )


BM = 1024
GRID = (N + BM - 1) // BM

_full = lambda i: (0, 0)


def _dinv_from(degp):
    return lax.rsqrt(1.0 + jnp.sum(degp, axis=0))[:, None]


def _tc1_body(x_ref, w1_ref, degp_ref, hs_ref):
    dinv = _dinv_from(degp_ref[...])
    r = jnp.dot(x_ref[...], w1_ref[...], preferred_element_type=jnp.float32,
                precision=lax.Precision.DEFAULT)
    r = r * dinv
    hs_ref[0] = r[:, :DH]
    hs_ref[1] = r[:, DH:]


def _ln_relu(t, g, b):
    mu = jnp.mean(t, axis=1, keepdims=True)
    var = jnp.mean((t - mu) ** 2, axis=1, keepdims=True)
    return jnp.maximum((t - mu) * lax.rsqrt(var + EPS) * g + b, 0.0)


def _tc2_body(acc_ref, hs_ref, degp_ref, w2_ref, b1_ref, g1_ref, bb1_ref,
              h1_ref, hs2_ref):
    dinv = _dinv_from(degp_ref[...])
    acc = jnp.concatenate([acc_ref[0], acc_ref[1]], axis=1)
    hs = jnp.concatenate([hs_ref[0], hs_ref[1]], axis=1)
    t = dinv * (acc + hs) + b1_ref[...]
    h1 = _ln_relu(t, g1_ref[...], bb1_ref[...])
    h1_ref[...] = h1
    r = jnp.dot(h1, w2_ref[...], preferred_element_type=jnp.float32,
                precision=lax.Precision.DEFAULT)
    r = r * dinv
    hs2_ref[0] = r[:, :DH]
    hs2_ref[1] = r[:, DH:]


def _tc3a_body(x_ref, h1_ref, fc1ab_ref, fc1b_ref, u0_ref):
    fc1ab = fc1ab_ref[...]
    dot = functools.partial(jnp.dot, preferred_element_type=jnp.float32,
                            precision=lax.Precision.DEFAULT)
    u0_ref[...] = (dot(x_ref[...], fc1ab[:D]) + dot(h1_ref[...], fc1ab[D:])
                   + fc1b_ref[...])


def _tc3b_body(u0_ref, acc_ref, hs_ref, degp_ref, b2_ref, g2_ref,
               bb2_ref, fc1c_ref, fc2w_ref, fc2b_ref, out_ref):
    dinv = _dinv_from(degp_ref[...])
    acc = jnp.concatenate([acc_ref[0], acc_ref[1]], axis=1)
    hs = jnp.concatenate([hs_ref[0], hs_ref[1]], axis=1)
    t = dinv * (acc + hs) + b2_ref[...]
    h2 = _ln_relu(t, g2_ref[...], bb2_ref[...])
    dot = functools.partial(jnp.dot, preferred_element_type=jnp.float32,
                            precision=lax.Precision.DEFAULT)
    u = jnp.maximum(u0_ref[...] + dot(h2, fc1c_ref[...]), 0.0)
    out_ref[...] = dot(u, fc2w_ref[...]) + fc2b_ref[...]


def _row_spec(width):
    return pl.BlockSpec((BM, width), lambda i: (i, 0))


_half_spec = pl.BlockSpec((2, BM, DH), lambda i: (0, i, 0))
_degp_spec = pl.BlockSpec((NTILES, BM), lambda i: (0, i))


def _vec_spec(width):
    return pl.BlockSpec((1, width), lambda i: (0, 0))


def _mat_spec(h, w):
    return pl.BlockSpec((h, w), _full)


def _tc1_call(x, W1, degp):
    return pl.pallas_call(
        _tc1_body,
        grid=(GRID,),
        in_specs=[_row_spec(D), _mat_spec(D, D), _degp_spec],
        out_specs=_half_spec,
        out_shape=jax.ShapeDtypeStruct((2, N, DH), jnp.float32),
    )(x, W1, degp)


def _tc2_call(acc1, hs1, degp, W2, b1, g1, bb1):
    return pl.pallas_call(
        _tc2_body,
        grid=(GRID,),
        in_specs=[_half_spec, _half_spec, _degp_spec, _mat_spec(D, D),
                  _vec_spec(D), _vec_spec(D), _vec_spec(D)],
        out_specs=[_row_spec(D), _half_spec],
        out_shape=[
            jax.ShapeDtypeStruct((N, D), jnp.float32),
            jax.ShapeDtypeStruct((2, N, DH), jnp.float32),
        ],
    )(acc1, hs1, degp, W2, b1, g1, bb1)


def _tc3a_call(x, h1, fc1_W, fc1_b):
    return pl.pallas_call(
        _tc3a_body,
        grid=(GRID,),
        in_specs=[_row_spec(D), _row_spec(D), _mat_spec(2 * D, D),
                  _vec_spec(D)],
        out_specs=_row_spec(D),
        out_shape=jax.ShapeDtypeStruct((N, D), jnp.float32),
    )(x, h1, fc1_W[:2 * D], fc1_b)


def _tc3b_call(u0, acc2, hs2, degp, b2, g2, bb2, fc1_W, fc2_W, fc2_b):
    return pl.pallas_call(
        _tc3b_body,
        grid=(GRID,),
        in_specs=[_row_spec(D), _half_spec, _half_spec,
                  _degp_spec, _vec_spec(D), _vec_spec(D), _vec_spec(D),
                  _mat_spec(D, D), _mat_spec(D, D), _vec_spec(D)],
        out_specs=_row_spec(D),
        out_shape=jax.ShapeDtypeStruct((N, D), jnp.float32),
    )(u0, acc2, hs2, degp, b2, g2, bb2, fc1_W[2 * D:], fc2_W, fc2_b)


def kernel(x, adj, weight, W1, b1, ln1_g, ln1_b, W2, b2, ln2_g, ln2_b,
           fc1_W, fc1_b, fc2_W, fc2_b):
    row = adj[0].astype(jnp.int32)
    col = adj[1].astype(jnp.int32)
    w = weight.astype(jnp.float32)

    b1r = b1.reshape(1, D)
    g1r = ln1_g.reshape(1, D)
    bb1r = ln1_b.reshape(1, D)
    b2r = b2.reshape(1, D)
    g2r = ln2_g.reshape(1, D)
    bb2r = ln2_b.reshape(1, D)
    fc1br = fc1_b.reshape(1, D)
    fc2br = fc2_b.reshape(1, D)

    degp = _deg_kernel(col, w).reshape(NTILES, N)
    row3 = row.reshape(NSUB, NCHUNK, CH)
    hs1 = _tc1_call(x, W1, degp)
    acc1 = _prop_kernel(hs1.reshape(2 * N, DH), row3, col, w)
    acc1 = acc1.reshape(2, N, DH)
    h1, hs2 = _tc2_call(acc1, hs1, degp, W2, b1r, g1r, bb1r)
    acc2 = _prop_kernel(hs2.reshape(2 * N, DH), row3, col, w)
    acc2 = acc2.reshape(2, N, DH)
    u0 = _tc3a_call(x, h1, fc1_W, fc1br)
    return _tc3b_call(u0, acc2, hs2, degp, b2r, g2r, bb2r,
                      fc1_W, fc2_W, fc2br)


# speedup vs baseline: 1.0700x; 1.0210x over previous
import functools

import jax
import jax.numpy as jnp
from jax import lax
from jax.experimental import pallas as pl
from jax.experimental.pallas import tpu as pltpu
from jax.experimental.pallas import tpu_sc as plsc

N = 10000
E = 160000
D = 256
DH = 128
EPS = 1e-5

NCORES = 2
NSUB = 16
NTILES = NCORES * NSUB

DEG_EPT = E // NTILES
PROP_EPT = E // NSUB
CH = 80
NCHUNK = PROP_EPT // CH
ZCH = CH

_SC_MESH = plsc.VectorSubcoreMesh(core_axis_name="c", subcore_axis_name="s")
_SC_PARAMS = pltpu.CompilerParams(needs_layout_passes=False)


def _zero16():
    return jnp.zeros((16,), jnp.float32)


@functools.partial(
    pl.kernel,
    mesh=_SC_MESH,
    compiler_params=_SC_PARAMS,
    out_type=jax.ShapeDtypeStruct((NTILES * N,), jnp.float32),
    scratch_types=[
        pltpu.VMEM((DEG_EPT + 16,), jnp.int32),
        pltpu.VMEM((DEG_EPT + 16,), jnp.float32),
        pltpu.VMEM((N,), jnp.float32),
    ],
)
def _deg_kernel(col_hbm, w_hbm, degp_hbm, colb, wb, degl):
    c = lax.axis_index("c")
    s = lax.axis_index("s")
    wid = s * NCORES + c
    base = wid * DEG_EPT

    pltpu.sync_copy(col_hbm.at[pl.ds(base, DEG_EPT)], colb.at[pl.ds(0, DEG_EPT)])
    pltpu.sync_copy(w_hbm.at[pl.ds(base, DEG_EPT)], wb.at[pl.ds(0, DEG_EPT)])

    def zbody(i, carry):
        degl[pl.ds(i * 16, 16)] = _zero16()
        return carry

    lax.fori_loop(0, N // 16, zbody, 0)

    nfull = DEG_EPT // 16
    rem = DEG_EPT - nfull * 16

    def sbody(j, carry):
        idx = colb[pl.ds(j * 16, 16)]
        vals = wb[pl.ds(j * 16, 16)]
        plsc.addupdate_scatter(degl, [idx], vals)
        return carry

    lax.fori_loop(0, nfull, sbody, 0)

    lane = jax.lax.broadcasted_iota(jnp.int32, (16,), 0)
    mask = lane < rem
    idx = colb[pl.ds(nfull * 16, 16)]
    idx = jnp.minimum(jnp.maximum(idx, 0), N - 1)
    vals = jnp.where(mask, wb[pl.ds(nfull * 16, 16)], 0.0)
    plsc.addupdate_scatter(degl, [idx], vals, mask=mask)

    pltpu.sync_copy(degl, degp_hbm.at[pl.ds(wid * N, N)])


@functools.partial(
    pl.kernel,
    mesh=_SC_MESH,
    compiler_params=_SC_PARAMS,
    out_type=jax.ShapeDtypeStruct((NCORES * N, DH), jnp.float32),
    scratch_types=[
        pltpu.VMEM_SHARED((N, DH), jnp.float32),
        pltpu.VMEM((NCHUNK, CH), jnp.int32),
        pltpu.VMEM((2, CH), jnp.float32),
        pltpu.VMEM((2, CH), jnp.int32),
        pltpu.VMEM((CH, DH), jnp.float32),
        pltpu.VMEM((CH, DH), jnp.float32),
        pltpu.SemaphoreType.DMA,
        pltpu.SemaphoreType.DMA,
        pltpu.SemaphoreType.DMA,
        pltpu.SemaphoreType.DMA,
        pltpu.SemaphoreType.DMA,
        pltpu.SemaphoreType.DMA,
        pltpu.SemaphoreType.DMA,
        pltpu.SemaphoreType.DMA,
    ],
)
def _prop_kernel(hsf_hbm, row_hbm, col_hbm, w_hbm, out_hbm, acc_sh,
                 rowb, wsm, colsm, rows0, rows1,
                 sg0, sg1, sc0, sc1, sw0, sw1, ss0, ss1):
    c = lax.axis_index("c")
    s = lax.axis_index("s")
    cN = c * N

    pltpu.sync_copy(row_hbm.at[s], rowb)

    def obody(i, carry):
        for v in range(CH // 16):
            rowb[i, pl.ds(v * 16, 16)] = rowb[i, pl.ds(v * 16, 16)] + cN
        return carry

    lax.fori_loop(0, NCHUNK, obody, 0)

    def zbody(i, carry):
        for f in range(DH // 16):
            rows0[i, pl.ds(f * 16, 16)] = _zero16()
        return carry

    lax.fori_loop(0, ZCH, zbody, 0)
    nrowchunks = N // ZCH
    for k in range((nrowchunks + NSUB - 1) // NSUB):
        j = s + k * NSUB

        @pl.when(j < nrowchunks)
        def _():
            pltpu.sync_copy(rows0, acc_sh.at[pl.ds(j * ZCH, ZCH)])

    plsc.subcore_barrier()

    def start(k, bufs):
        rbuf, semg, cbuf, semc, wbuf, semw, sems = bufs
        ebase = s * PROP_EPT + k * CH
        pltpu.async_copy(hsf_hbm.at[rowb.at[k]], rbuf, semg)
        pltpu.async_copy(col_hbm.at[pl.ds(ebase, CH)], cbuf, semc)
        pltpu.async_copy(w_hbm.at[pl.ds(ebase, CH)], wbuf, semw)

    def process(k, bufs, nbufs):
        rbuf, semg, cbuf, semc, wbuf, semw, sems = bufs

        @pl.when(k >= 1)
        def _():
            pltpu.make_async_copy(
                nbufs[0], acc_sh.at[pl.ds(0, CH)], nbufs[6]).wait()

        @pl.when(k + 1 < NCHUNK)
        def _():
            start(k + 1, nbufs)

        pltpu.make_async_copy(hsf_hbm.at[pl.ds(0, CH)], rbuf, semg).wait()
        pltpu.make_async_copy(w_hbm.at[pl.ds(0, CH)], wbuf, semw).wait()

        @plsc.parallel_loop(0, CH, 1, unroll=4)
        def scale(j):
            wv = plsc.load_gather(wbuf, [jnp.full((16,), j, jnp.int32)])
            for f in range(DH // 16):
                rbuf[j, pl.ds(f * 16, 16)] = rbuf[j, pl.ds(f * 16, 16)] * wv

        pltpu.make_async_copy(col_hbm.at[pl.ds(0, CH)], cbuf, semc).wait()
        pltpu.async_copy(rbuf, acc_sh.at[cbuf], sems, add=True)

    bufs0 = (rows0, sg0, colsm.at[0], sc0, wsm.at[0], sw0, ss0)
    bufs1 = (rows1, sg1, colsm.at[1], sc1, wsm.at[1], sw1, ss1)
    start(0, bufs0)

    def gbody(g, carry):
        process(2 * g, bufs0, bufs1)
        process(2 * g + 1, bufs1, bufs0)
        return carry

    lax.fori_loop(0, (NCHUNK - 1) // 2, gbody, 0)
    process(NCHUNK - 1, bufs0, bufs1)
    pltpu.make_async_copy(rows0, acc_sh.at[pl.ds(0, CH)], ss0).wait()

    plsc.subcore_barrier()

    for k in range((nrowchunks + NSUB - 1) // NSUB):
        j = s + k * NSUB

        @pl.when(j < nrowchunks)
        def _():
            pltpu.sync_copy(
                acc_sh.at[pl.ds(j * ZCH, ZCH)],
                out_hbm.at[pl.ds(c * N + j * ZCH, ZCH)],
            )


BM = 2048
GRID = (N + BM - 1) // BM

_full = lambda i: (0, 0)


def _dinv_from(degp):
    return lax.rsqrt(1.0 + jnp.sum(degp, axis=0))[:, None]


def _tc1_body(x_ref, w1_ref, degp_ref, hs_ref):
    dinv = _dinv_from(degp_ref[...])
    r = jnp.dot(x_ref[...], w1_ref[...], preferred_element_type=jnp.float32,
                precision=lax.Precision.DEFAULT)
    r = r * dinv
    hs_ref[0] = r[:, :DH]
    hs_ref[1] = r[:, DH:]


def _ln_relu(t, g, b):
    mu = jnp.mean(t, axis=1, keepdims=True)
    var = jnp.mean((t - mu) ** 2, axis=1, keepdims=True)
    return jnp.maximum((t - mu) * lax.rsqrt(var + EPS) * g + b, 0.0)


def _tc2_body(acc_ref, hs_ref, degp_ref, w2_ref, b1_ref, g1_ref, bb1_ref,
              h1_ref, hs2_ref):
    dinv = _dinv_from(degp_ref[...])
    acc = jnp.concatenate([acc_ref[0], acc_ref[1]], axis=1)
    hs = jnp.concatenate([hs_ref[0], hs_ref[1]], axis=1)
    t = dinv * (acc + hs) + b1_ref[...]
    h1 = _ln_relu(t, g1_ref[...], bb1_ref[...])
    h1_ref[...] = h1
    r = jnp.dot(h1, w2_ref[...], preferred_element_type=jnp.float32,
                precision=lax.Precision.DEFAULT)
    r = r * dinv
    hs2_ref[0] = r[:, :DH]
    hs2_ref[1] = r[:, DH:]


def _tc3a_body(x_ref, h1_ref, fc1ab_ref, fc1b_ref, u0_ref):
    fc1ab = fc1ab_ref[...]
    dot = functools.partial(jnp.dot, preferred_element_type=jnp.float32,
                            precision=lax.Precision.DEFAULT)
    u0_ref[...] = (dot(x_ref[...], fc1ab[:D]) + dot(h1_ref[...], fc1ab[D:])
                   + fc1b_ref[...])


def _tc3b_body(u0_ref, acc_ref, hs_ref, degp_ref, b2_ref, g2_ref,
               bb2_ref, fc1c_ref, fc2w_ref, fc2b_ref, out_ref):
    dinv = _dinv_from(degp_ref[...])
    acc = jnp.concatenate([acc_ref[0], acc_ref[1]], axis=1)
    hs = jnp.concatenate([hs_ref[0], hs_ref[1]], axis=1)
    t = dinv * (acc + hs) + b2_ref[...]
    h2 = _ln_relu(t, g2_ref[...], bb2_ref[...])
    dot = functools.partial(jnp.dot, preferred_element_type=jnp.float32,
                            precision=lax.Precision.DEFAULT)
    u = jnp.maximum(u0_ref[...] + dot(h2, fc1c_ref[...]), 0.0)
    out_ref[...] = dot(u, fc2w_ref[...]) + fc2b_ref[...]


def _row_spec(width):
    return pl.BlockSpec((BM, width), lambda i: (i, 0))


_half_spec = pl.BlockSpec((2, BM, DH), lambda i: (0, i, 0))
_degp_spec = pl.BlockSpec((NTILES, BM), lambda i: (0, i))


def _vec_spec(width):
    return pl.BlockSpec((1, width), lambda i: (0, 0))


def _mat_spec(h, w):
    return pl.BlockSpec((h, w), _full)


def _tc1_call(x, W1, degp):
    return pl.pallas_call(
        _tc1_body,
        grid=(GRID,),
        in_specs=[_row_spec(D), _mat_spec(D, D), _degp_spec],
        out_specs=_half_spec,
        out_shape=jax.ShapeDtypeStruct((2, N, DH), jnp.float32),
    )(x, W1, degp)


def _tc2_call(acc1, hs1, degp, W2, b1, g1, bb1):
    return pl.pallas_call(
        _tc2_body,
        grid=(GRID,),
        in_specs=[_half_spec, _half_spec, _degp_spec, _mat_spec(D, D),
                  _vec_spec(D), _vec_spec(D), _vec_spec(D)],
        out_specs=[_row_spec(D), _half_spec],
        out_shape=[
            jax.ShapeDtypeStruct((N, D), jnp.float32),
            jax.ShapeDtypeStruct((2, N, DH), jnp.float32),
        ],
    )(acc1, hs1, degp, W2, b1, g1, bb1)


def _tc3a_call(x, h1, fc1_W, fc1_b):
    return pl.pallas_call(
        _tc3a_body,
        grid=(GRID,),
        in_specs=[_row_spec(D), _row_spec(D), _mat_spec(2 * D, D),
                  _vec_spec(D)],
        out_specs=_row_spec(D),
        out_shape=jax.ShapeDtypeStruct((N, D), jnp.float32),
    )(x, h1, fc1_W[:2 * D], fc1_b)


def _tc3b_call(u0, acc2, hs2, degp, b2, g2, bb2, fc1_W, fc2_W, fc2_b):
    return pl.pallas_call(
        _tc3b_body,
        grid=(GRID,),
        in_specs=[_row_spec(D), _half_spec, _half_spec,
                  _degp_spec, _vec_spec(D), _vec_spec(D), _vec_spec(D),
                  _mat_spec(D, D), _mat_spec(D, D), _vec_spec(D)],
        out_specs=_row_spec(D),
        out_shape=jax.ShapeDtypeStruct((N, D), jnp.float32),
    )(u0, acc2, hs2, degp, b2, g2, bb2, fc1_W[2 * D:], fc2_W, fc2_b)


def kernel(x, adj, weight, W1, b1, ln1_g, ln1_b, W2, b2, ln2_g, ln2_b,
           fc1_W, fc1_b, fc2_W, fc2_b):
    row = adj[0].astype(jnp.int32)
    col = adj[1].astype(jnp.int32)
    w = weight.astype(jnp.float32)

    b1r = b1.reshape(1, D)
    g1r = ln1_g.reshape(1, D)
    bb1r = ln1_b.reshape(1, D)
    b2r = b2.reshape(1, D)
    g2r = ln2_g.reshape(1, D)
    bb2r = ln2_b.reshape(1, D)
    fc1br = fc1_b.reshape(1, D)
    fc2br = fc2_b.reshape(1, D)

    degp = _deg_kernel(col, w).reshape(NTILES, N)
    row3 = row.reshape(NSUB, NCHUNK, CH)
    hs1 = _tc1_call(x, W1, degp)
    acc1 = _prop_kernel(hs1.reshape(2 * N, DH), row3, col, w)
    acc1 = acc1.reshape(2, N, DH)
    h1, hs2 = _tc2_call(acc1, hs1, degp, W2, b1r, g1r, bb1r)
    acc2 = _prop_kernel(hs2.reshape(2 * N, DH), row3, col, w)
    acc2 = acc2.reshape(2, N, DH)
    u0 = _tc3a_call(x, h1, fc1_W, fc1br)
    return _tc3b_call(u0, acc2, hs2, degp, b2r, g2r, bb2r,
                      fc1_W, fc2_W, fc2br)


# speedup vs baseline: 1.0812x; 1.0105x over previous
import functools

import jax
import jax.numpy as jnp
from jax import lax
from jax.experimental import pallas as pl
from jax.experimental.pallas import tpu as pltpu
from jax.experimental.pallas import tpu_sc as plsc

N = 10000
E = 160000
D = 256
DH = 128
EPS = 1e-5

NCORES = 2
NSUB = 16
NTILES = NCORES * NSUB

DEG_EPT = E // NTILES
PROP_EPT = E // NSUB
CH = 80
NCHUNK = PROP_EPT // CH
ZCH = CH

_SC_MESH = plsc.VectorSubcoreMesh(core_axis_name="c", subcore_axis_name="s")
_SC_PARAMS = pltpu.CompilerParams(needs_layout_passes=False)


def _zero16():
    return jnp.zeros((16,), jnp.float32)


@functools.partial(
    pl.kernel,
    mesh=_SC_MESH,
    compiler_params=_SC_PARAMS,
    out_type=jax.ShapeDtypeStruct((NTILES * N,), jnp.float32),
    scratch_types=[
        pltpu.VMEM((DEG_EPT + 16,), jnp.int32),
        pltpu.VMEM((DEG_EPT + 16,), jnp.float32),
        pltpu.VMEM((N,), jnp.float32),
    ],
)
def _deg_kernel(col_hbm, w_hbm, degp_hbm, colb, wb, degl):
    c = lax.axis_index("c")
    s = lax.axis_index("s")
    wid = s * NCORES + c
    base = wid * DEG_EPT

    pltpu.sync_copy(col_hbm.at[pl.ds(base, DEG_EPT)], colb.at[pl.ds(0, DEG_EPT)])
    pltpu.sync_copy(w_hbm.at[pl.ds(base, DEG_EPT)], wb.at[pl.ds(0, DEG_EPT)])

    def zbody(i, carry):
        degl[pl.ds(i * 16, 16)] = _zero16()
        return carry

    lax.fori_loop(0, N // 16, zbody, 0)

    nfull = DEG_EPT // 16
    rem = DEG_EPT - nfull * 16

    def sbody(j, carry):
        idx = colb[pl.ds(j * 16, 16)]
        vals = wb[pl.ds(j * 16, 16)]
        plsc.addupdate_scatter(degl, [idx], vals)
        return carry

    lax.fori_loop(0, nfull, sbody, 0)

    lane = jax.lax.broadcasted_iota(jnp.int32, (16,), 0)
    mask = lane < rem
    idx = colb[pl.ds(nfull * 16, 16)]
    idx = jnp.minimum(jnp.maximum(idx, 0), N - 1)
    vals = jnp.where(mask, wb[pl.ds(nfull * 16, 16)], 0.0)
    plsc.addupdate_scatter(degl, [idx], vals, mask=mask)

    pltpu.sync_copy(degl, degp_hbm.at[pl.ds(wid * N, N)])


@functools.partial(
    pl.kernel,
    mesh=_SC_MESH,
    compiler_params=_SC_PARAMS,
    out_type=jax.ShapeDtypeStruct((NCORES * N, DH), jnp.float32),
    scratch_types=[
        pltpu.VMEM_SHARED((N, DH), jnp.float32),
        pltpu.VMEM((NCHUNK, CH), jnp.int32),
        pltpu.VMEM((2, CH), jnp.float32),
        pltpu.VMEM((2, CH), jnp.int32),
        pltpu.VMEM((CH, DH), jnp.float32),
        pltpu.VMEM((CH, DH), jnp.float32),
        pltpu.SemaphoreType.DMA,
        pltpu.SemaphoreType.DMA,
        pltpu.SemaphoreType.DMA,
        pltpu.SemaphoreType.DMA,
        pltpu.SemaphoreType.DMA,
        pltpu.SemaphoreType.DMA,
        pltpu.SemaphoreType.DMA,
        pltpu.SemaphoreType.DMA,
    ],
)
def _prop_kernel(hsf_hbm, row_hbm, col_hbm, w_hbm, out_hbm, acc_sh,
                 rowb, wsm, colsm, rows0, rows1,
                 sg0, sg1, sc0, sc1, sw0, sw1, ss0, ss1):
    c = lax.axis_index("c")
    s = lax.axis_index("s")
    cN = c * N

    pltpu.sync_copy(row_hbm.at[s], rowb)

    def obody(i, carry):
        for v in range(CH // 16):
            rowb[i, pl.ds(v * 16, 16)] = rowb[i, pl.ds(v * 16, 16)] + cN
        return carry

    lax.fori_loop(0, NCHUNK, obody, 0)

    def zbody(i, carry):
        for f in range(DH // 16):
            rows0[i, pl.ds(f * 16, 16)] = _zero16()
        return carry

    lax.fori_loop(0, ZCH, zbody, 0)
    nrowchunks = N // ZCH
    for k in range((nrowchunks + NSUB - 1) // NSUB):
        j = s + k * NSUB

        @pl.when(j < nrowchunks)
        def _():
            pltpu.sync_copy(rows0, acc_sh.at[pl.ds(j * ZCH, ZCH)])

    plsc.subcore_barrier()

    def start(k, bufs):
        rbuf, semg, cbuf, semc, wbuf, semw, sems = bufs
        ebase = s * PROP_EPT + k * CH
        pltpu.async_copy(hsf_hbm.at[rowb.at[k]], rbuf, semg)
        pltpu.async_copy(col_hbm.at[pl.ds(ebase, CH)], cbuf, semc)
        pltpu.async_copy(w_hbm.at[pl.ds(ebase, CH)], wbuf, semw)

    def process(k, bufs, nbufs):
        rbuf, semg, cbuf, semc, wbuf, semw, sems = bufs

        @pl.when(k >= 1)
        def _():
            pltpu.make_async_copy(
                nbufs[0], acc_sh.at[pl.ds(0, CH)], nbufs[6]).wait()

        @pl.when(k + 1 < NCHUNK)
        def _():
            start(k + 1, nbufs)

        pltpu.make_async_copy(hsf_hbm.at[pl.ds(0, CH)], rbuf, semg).wait()
        pltpu.make_async_copy(w_hbm.at[pl.ds(0, CH)], wbuf, semw).wait()

        @plsc.parallel_loop(0, CH, 1, unroll=4)
        def scale(j):
            wv = plsc.load_gather(wbuf, [jnp.full((16,), j, jnp.int32)])
            for f in range(DH // 16):
                rbuf[j, pl.ds(f * 16, 16)] = rbuf[j, pl.ds(f * 16, 16)] * wv

        pltpu.make_async_copy(col_hbm.at[pl.ds(0, CH)], cbuf, semc).wait()
        pltpu.async_copy(rbuf, acc_sh.at[cbuf], sems, add=True)

    bufs0 = (rows0, sg0, colsm.at[0], sc0, wsm.at[0], sw0, ss0)
    bufs1 = (rows1, sg1, colsm.at[1], sc1, wsm.at[1], sw1, ss1)
    start(0, bufs0)

    def gbody(g, carry):
        process(2 * g, bufs0, bufs1)
        process(2 * g + 1, bufs1, bufs0)
        return carry

    lax.fori_loop(0, (NCHUNK - 1) // 2, gbody, 0)
    process(NCHUNK - 1, bufs0, bufs1)
    pltpu.make_async_copy(rows0, acc_sh.at[pl.ds(0, CH)], ss0).wait()

    plsc.subcore_barrier()

    for k in range((nrowchunks + NSUB - 1) // NSUB):
        j = s + k * NSUB

        @pl.when(j < nrowchunks)
        def _():
            pltpu.sync_copy(
                acc_sh.at[pl.ds(j * ZCH, ZCH)],
                out_hbm.at[pl.ds(c * N + j * ZCH, ZCH)],
            )


BM = 4096
GRID = (N + BM - 1) // BM

_full = lambda i: (0, 0)


def _dinv_from(degp):
    return lax.rsqrt(1.0 + jnp.sum(degp, axis=0))[:, None]


def _tc1_body(x_ref, w1_ref, degp_ref, hs_ref):
    dinv = _dinv_from(degp_ref[...])
    r = jnp.dot(x_ref[...], w1_ref[...], preferred_element_type=jnp.float32,
                precision=lax.Precision.DEFAULT)
    r = r * dinv
    hs_ref[0] = r[:, :DH]
    hs_ref[1] = r[:, DH:]


def _ln_relu(t, g, b):
    mu = jnp.mean(t, axis=1, keepdims=True)
    var = jnp.mean((t - mu) ** 2, axis=1, keepdims=True)
    return jnp.maximum((t - mu) * lax.rsqrt(var + EPS) * g + b, 0.0)


def _tc2_body(acc_ref, hs_ref, degp_ref, w2_ref, b1_ref, g1_ref, bb1_ref,
              h1_ref, hs2_ref):
    dinv = _dinv_from(degp_ref[...])
    acc = jnp.concatenate([acc_ref[0], acc_ref[1]], axis=1)
    hs = jnp.concatenate([hs_ref[0], hs_ref[1]], axis=1)
    t = dinv * (acc + hs) + b1_ref[...]
    h1 = _ln_relu(t, g1_ref[...], bb1_ref[...])
    h1_ref[...] = h1
    r = jnp.dot(h1, w2_ref[...], preferred_element_type=jnp.float32,
                precision=lax.Precision.DEFAULT)
    r = r * dinv
    hs2_ref[0] = r[:, :DH]
    hs2_ref[1] = r[:, DH:]


def _tc3a_body(x_ref, h1_ref, fc1ab_ref, fc1b_ref, u0_ref):
    fc1ab = fc1ab_ref[...]
    dot = functools.partial(jnp.dot, preferred_element_type=jnp.float32,
                            precision=lax.Precision.DEFAULT)
    u0_ref[...] = (dot(x_ref[...], fc1ab[:D]) + dot(h1_ref[...], fc1ab[D:])
                   + fc1b_ref[...])


def _tc3b_body(u0_ref, acc_ref, hs_ref, degp_ref, b2_ref, g2_ref,
               bb2_ref, fc1c_ref, fc2w_ref, fc2b_ref, out_ref):
    dinv = _dinv_from(degp_ref[...])
    acc = jnp.concatenate([acc_ref[0], acc_ref[1]], axis=1)
    hs = jnp.concatenate([hs_ref[0], hs_ref[1]], axis=1)
    t = dinv * (acc + hs) + b2_ref[...]
    h2 = _ln_relu(t, g2_ref[...], bb2_ref[...])
    dot = functools.partial(jnp.dot, preferred_element_type=jnp.float32,
                            precision=lax.Precision.DEFAULT)
    u = jnp.maximum(u0_ref[...] + dot(h2, fc1c_ref[...]), 0.0)
    out_ref[...] = dot(u, fc2w_ref[...]) + fc2b_ref[...]


def _row_spec(width):
    return pl.BlockSpec((BM, width), lambda i: (i, 0))


_half_spec = pl.BlockSpec((2, BM, DH), lambda i: (0, i, 0))
_degp_spec = pl.BlockSpec((NTILES, BM), lambda i: (0, i))


def _vec_spec(width):
    return pl.BlockSpec((1, width), lambda i: (0, 0))


def _mat_spec(h, w):
    return pl.BlockSpec((h, w), _full)


def _tc1_call(x, W1, degp):
    return pl.pallas_call(
        _tc1_body,
        grid=(GRID,),
        in_specs=[_row_spec(D), _mat_spec(D, D), _degp_spec],
        out_specs=_half_spec,
        out_shape=jax.ShapeDtypeStruct((2, N, DH), jnp.float32),
    )(x, W1, degp)


def _tc2_call(acc1, hs1, degp, W2, b1, g1, bb1):
    return pl.pallas_call(
        _tc2_body,
        grid=(GRID,),
        in_specs=[_half_spec, _half_spec, _degp_spec, _mat_spec(D, D),
                  _vec_spec(D), _vec_spec(D), _vec_spec(D)],
        out_specs=[_row_spec(D), _half_spec],
        out_shape=[
            jax.ShapeDtypeStruct((N, D), jnp.float32),
            jax.ShapeDtypeStruct((2, N, DH), jnp.float32),
        ],
    )(acc1, hs1, degp, W2, b1, g1, bb1)


def _tc3a_call(x, h1, fc1_W, fc1_b):
    return pl.pallas_call(
        _tc3a_body,
        grid=(GRID,),
        in_specs=[_row_spec(D), _row_spec(D), _mat_spec(2 * D, D),
                  _vec_spec(D)],
        out_specs=_row_spec(D),
        out_shape=jax.ShapeDtypeStruct((N, D), jnp.float32),
    )(x, h1, fc1_W[:2 * D], fc1_b)


def _tc3b_call(u0, acc2, hs2, degp, b2, g2, bb2, fc1_W, fc2_W, fc2_b):
    return pl.pallas_call(
        _tc3b_body,
        grid=(GRID,),
        in_specs=[_row_spec(D), _half_spec, _half_spec,
                  _degp_spec, _vec_spec(D), _vec_spec(D), _vec_spec(D),
                  _mat_spec(D, D), _mat_spec(D, D), _vec_spec(D)],
        out_specs=_row_spec(D),
        out_shape=jax.ShapeDtypeStruct((N, D), jnp.float32),
    )(u0, acc2, hs2, degp, b2, g2, bb2, fc1_W[2 * D:], fc2_W, fc2_b)


def kernel(x, adj, weight, W1, b1, ln1_g, ln1_b, W2, b2, ln2_g, ln2_b,
           fc1_W, fc1_b, fc2_W, fc2_b):
    row = adj[0].astype(jnp.int32)
    col = adj[1].astype(jnp.int32)
    w = weight.astype(jnp.float32)

    b1r = b1.reshape(1, D)
    g1r = ln1_g.reshape(1, D)
    bb1r = ln1_b.reshape(1, D)
    b2r = b2.reshape(1, D)
    g2r = ln2_g.reshape(1, D)
    bb2r = ln2_b.reshape(1, D)
    fc1br = fc1_b.reshape(1, D)
    fc2br = fc2_b.reshape(1, D)

    degp = _deg_kernel(col, w).reshape(NTILES, N)
    row3 = row.reshape(NSUB, NCHUNK, CH)
    hs1 = _tc1_call(x, W1, degp)
    acc1 = _prop_kernel(hs1.reshape(2 * N, DH), row3, col, w)
    acc1 = acc1.reshape(2, N, DH)
    h1, hs2 = _tc2_call(acc1, hs1, degp, W2, b1r, g1r, bb1r)
    acc2 = _prop_kernel(hs2.reshape(2 * N, DH), row3, col, w)
    acc2 = acc2.reshape(2, N, DH)
    u0 = _tc3a_call(x, h1, fc1_W, fc1br)
    return _tc3b_call(u0, acc2, hs2, degp, b2r, g2r, bb2r,
                      fc1_W, fc2_W, fc2br)


# speedup vs baseline: 1.1166x; 1.0327x over previous
import functools

import jax
import jax.numpy as jnp
from jax import lax
from jax.experimental import pallas as pl
from jax.experimental.pallas import tpu as pltpu
from jax.experimental.pallas import tpu_sc as plsc

N = 10000
E = 160000
D = 256
DH = 128
EPS = 1e-5

NCORES = 2
NSUB = 16
NTILES = NCORES * NSUB

DEG_EPT = E // NTILES
PROP_EPT = E // NSUB
CH = 80
NCHUNK = PROP_EPT // CH
ZCH = CH

_SC_MESH = plsc.VectorSubcoreMesh(core_axis_name="c", subcore_axis_name="s")
_SC_PARAMS = pltpu.CompilerParams(needs_layout_passes=False)


def _zero16():
    return jnp.zeros((16,), jnp.float32)


@functools.partial(
    pl.kernel,
    mesh=_SC_MESH,
    compiler_params=_SC_PARAMS,
    out_type=jax.ShapeDtypeStruct((NTILES * N,), jnp.float32),
    scratch_types=[
        pltpu.VMEM((DEG_EPT + 16,), jnp.int32),
        pltpu.VMEM((DEG_EPT + 16,), jnp.float32),
        pltpu.VMEM((N,), jnp.float32),
    ],
)
def _deg_kernel(col_hbm, w_hbm, degp_hbm, colb, wb, degl):
    c = lax.axis_index("c")
    s = lax.axis_index("s")
    wid = s * NCORES + c
    base = wid * DEG_EPT

    pltpu.sync_copy(col_hbm.at[pl.ds(base, DEG_EPT)], colb.at[pl.ds(0, DEG_EPT)])
    pltpu.sync_copy(w_hbm.at[pl.ds(base, DEG_EPT)], wb.at[pl.ds(0, DEG_EPT)])

    def zbody(i, carry):
        degl[pl.ds(i * 16, 16)] = _zero16()
        return carry

    lax.fori_loop(0, N // 16, zbody, 0)

    nfull = DEG_EPT // 16
    rem = DEG_EPT - nfull * 16

    def sbody(j, carry):
        idx = colb[pl.ds(j * 16, 16)]
        vals = wb[pl.ds(j * 16, 16)]
        plsc.addupdate_scatter(degl, [idx], vals)
        return carry

    lax.fori_loop(0, nfull, sbody, 0)

    lane = jax.lax.broadcasted_iota(jnp.int32, (16,), 0)
    mask = lane < rem
    idx = colb[pl.ds(nfull * 16, 16)]
    idx = jnp.minimum(jnp.maximum(idx, 0), N - 1)
    vals = jnp.where(mask, wb[pl.ds(nfull * 16, 16)], 0.0)
    plsc.addupdate_scatter(degl, [idx], vals, mask=mask)

    pltpu.sync_copy(degl, degp_hbm.at[pl.ds(wid * N, N)])


@functools.partial(
    pl.kernel,
    mesh=_SC_MESH,
    compiler_params=_SC_PARAMS,
    out_type=jax.ShapeDtypeStruct((NCORES * N, DH), jnp.float32),
    scratch_types=[
        pltpu.VMEM_SHARED((N, DH), jnp.float32),
        pltpu.VMEM((NCHUNK, CH), jnp.int32),
        pltpu.VMEM((2, CH), jnp.float32),
        pltpu.VMEM((4, CH // 2), jnp.int32),
        pltpu.VMEM((CH, DH), jnp.float32),
        pltpu.VMEM((CH, DH), jnp.float32),
        pltpu.SemaphoreType.DMA,
        pltpu.SemaphoreType.DMA,
        pltpu.SemaphoreType.DMA,
        pltpu.SemaphoreType.DMA,
        pltpu.SemaphoreType.DMA,
        pltpu.SemaphoreType.DMA,
        pltpu.SemaphoreType.DMA,
        pltpu.SemaphoreType.DMA,
    ],
)
def _prop_kernel(hsf_hbm, row_hbm, col_hbm, w_hbm, out_hbm, acc_sh,
                 rowb, wsm, colsm, rows0, rows1,
                 sg0, sg1, sc0, sc1, sw0, sw1, ss0, ss1):
    HC = CH // 2
    c = lax.axis_index("c")
    s = lax.axis_index("s")
    cN = c * N

    pltpu.sync_copy(row_hbm.at[s], rowb)

    def obody(i, carry):
        for v in range(CH // 16):
            rowb[i, pl.ds(v * 16, 16)] = rowb[i, pl.ds(v * 16, 16)] + cN
        return carry

    lax.fori_loop(0, NCHUNK, obody, 0)

    def zbody(i, carry):
        for f in range(DH // 16):
            rows0[i, pl.ds(f * 16, 16)] = _zero16()
        return carry

    lax.fori_loop(0, ZCH, zbody, 0)
    nrowchunks = N // ZCH
    for k in range((nrowchunks + NSUB - 1) // NSUB):
        j = s + k * NSUB

        @pl.when(j < nrowchunks)
        def _():
            pltpu.sync_copy(rows0, acc_sh.at[pl.ds(j * ZCH, ZCH)])

    plsc.subcore_barrier()

    def start(k, bufs):
        rbuf, semg, cqA, cqB, semc, wbuf, semw, sems = bufs
        ebase = s * PROP_EPT + k * CH
        pltpu.async_copy(hsf_hbm.at[rowb.at[k]], rbuf, semg)
        pltpu.async_copy(col_hbm.at[pl.ds(ebase, HC)], cqA, semc)
        pltpu.async_copy(col_hbm.at[pl.ds(ebase + HC, HC)], cqB, semc)
        pltpu.async_copy(w_hbm.at[pl.ds(ebase, CH)], wbuf, semw)

    def process(k, bufs, nbufs):
        rbuf, semg, cqA, cqB, semc, wbuf, semw, sems = bufs

        @pl.when(k >= 1)
        def _():
            pltpu.make_async_copy(
                nbufs[0].at[pl.ds(0, HC)], acc_sh.at[pl.ds(0, HC)],
                nbufs[7]).wait()
            pltpu.make_async_copy(
                nbufs[0].at[pl.ds(0, HC)], acc_sh.at[pl.ds(0, HC)],
                nbufs[7]).wait()

        @pl.when(k + 1 < NCHUNK)
        def _():
            start(k + 1, nbufs)

        pltpu.make_async_copy(hsf_hbm.at[pl.ds(0, CH)], rbuf, semg).wait()
        pltpu.make_async_copy(w_hbm.at[pl.ds(0, CH)], wbuf, semw).wait()

        def scale_half(joff):
            @plsc.parallel_loop(0, HC, 1, unroll=4)
            def scale(j):
                je = j + joff
                wv = plsc.load_gather(wbuf, [jnp.full((16,), je, jnp.int32)])
                for f in range(DH // 16):
                    rbuf[je, pl.ds(f * 16, 16)] = (
                        rbuf[je, pl.ds(f * 16, 16)] * wv)

        scale_half(0)
        pltpu.make_async_copy(col_hbm.at[pl.ds(0, HC)], cqA, semc).wait()
        pltpu.async_copy(rbuf.at[pl.ds(0, HC)], acc_sh.at[cqA], sems,
                         add=True)
        scale_half(HC)
        pltpu.make_async_copy(col_hbm.at[pl.ds(0, HC)], cqB, semc).wait()
        pltpu.async_copy(rbuf.at[pl.ds(HC, HC)], acc_sh.at[cqB], sems,
                         add=True)

    bufs0 = (rows0, sg0, colsm.at[0], colsm.at[1], sc0, wsm.at[0], sw0, ss0)
    bufs1 = (rows1, sg1, colsm.at[2], colsm.at[3], sc1, wsm.at[1], sw1, ss1)
    start(0, bufs0)

    def gbody(g, carry):
        process(2 * g, bufs0, bufs1)
        process(2 * g + 1, bufs1, bufs0)
        return carry

    lax.fori_loop(0, (NCHUNK - 1) // 2, gbody, 0)
    process(NCHUNK - 1, bufs0, bufs1)
    pltpu.make_async_copy(rows0.at[pl.ds(0, HC)], acc_sh.at[pl.ds(0, HC)],
                          ss0).wait()
    pltpu.make_async_copy(rows0.at[pl.ds(0, HC)], acc_sh.at[pl.ds(0, HC)],
                          ss0).wait()

    plsc.subcore_barrier()

    for k in range((nrowchunks + NSUB - 1) // NSUB):
        j = s + k * NSUB

        @pl.when(j < nrowchunks)
        def _():
            pltpu.sync_copy(
                acc_sh.at[pl.ds(j * ZCH, ZCH)],
                out_hbm.at[pl.ds(c * N + j * ZCH, ZCH)],
            )


BM = 4096
GRID = (N + BM - 1) // BM

_full = lambda i: (0, 0)


def _dinv_from(degp):
    return lax.rsqrt(1.0 + jnp.sum(degp, axis=0))[:, None]


def _tc1_body(x_ref, w1_ref, degp_ref, hs_ref):
    dinv = _dinv_from(degp_ref[...])
    r = jnp.dot(x_ref[...], w1_ref[...], preferred_element_type=jnp.float32,
                precision=lax.Precision.DEFAULT)
    r = r * dinv
    hs_ref[0] = r[:, :DH]
    hs_ref[1] = r[:, DH:]


def _ln_relu(t, g, b):
    mu = jnp.mean(t, axis=1, keepdims=True)
    var = jnp.mean((t - mu) ** 2, axis=1, keepdims=True)
    return jnp.maximum((t - mu) * lax.rsqrt(var + EPS) * g + b, 0.0)


def _tc2_body(acc_ref, hs_ref, degp_ref, w2_ref, b1_ref, g1_ref, bb1_ref,
              h1_ref, hs2_ref):
    dinv = _dinv_from(degp_ref[...])
    acc = jnp.concatenate([acc_ref[0], acc_ref[1]], axis=1)
    hs = jnp.concatenate([hs_ref[0], hs_ref[1]], axis=1)
    t = dinv * (acc + hs) + b1_ref[...]
    h1 = _ln_relu(t, g1_ref[...], bb1_ref[...])
    h1_ref[...] = h1
    r = jnp.dot(h1, w2_ref[...], preferred_element_type=jnp.float32,
                precision=lax.Precision.DEFAULT)
    r = r * dinv
    hs2_ref[0] = r[:, :DH]
    hs2_ref[1] = r[:, DH:]


def _tc3a_body(x_ref, h1_ref, fc1ab_ref, fc1b_ref, u0_ref):
    fc1ab = fc1ab_ref[...]
    dot = functools.partial(jnp.dot, preferred_element_type=jnp.float32,
                            precision=lax.Precision.DEFAULT)
    u0_ref[...] = (dot(x_ref[...], fc1ab[:D]) + dot(h1_ref[...], fc1ab[D:])
                   + fc1b_ref[...])


def _tc3b_body(u0_ref, acc_ref, hs_ref, degp_ref, b2_ref, g2_ref,
               bb2_ref, fc1c_ref, fc2w_ref, fc2b_ref, out_ref):
    dinv = _dinv_from(degp_ref[...])
    acc = jnp.concatenate([acc_ref[0], acc_ref[1]], axis=1)
    hs = jnp.concatenate([hs_ref[0], hs_ref[1]], axis=1)
    t = dinv * (acc + hs) + b2_ref[...]
    h2 = _ln_relu(t, g2_ref[...], bb2_ref[...])
    dot = functools.partial(jnp.dot, preferred_element_type=jnp.float32,
                            precision=lax.Precision.DEFAULT)
    u = jnp.maximum(u0_ref[...] + dot(h2, fc1c_ref[...]), 0.0)
    out_ref[...] = dot(u, fc2w_ref[...]) + fc2b_ref[...]


def _row_spec(width):
    return pl.BlockSpec((BM, width), lambda i: (i, 0))


_half_spec = pl.BlockSpec((2, BM, DH), lambda i: (0, i, 0))
_degp_spec = pl.BlockSpec((NTILES, BM), lambda i: (0, i))


def _vec_spec(width):
    return pl.BlockSpec((1, width), lambda i: (0, 0))


def _mat_spec(h, w):
    return pl.BlockSpec((h, w), _full)


def _tc1_call(x, W1, degp):
    return pl.pallas_call(
        _tc1_body,
        grid=(GRID,),
        in_specs=[_row_spec(D), _mat_spec(D, D), _degp_spec],
        out_specs=_half_spec,
        out_shape=jax.ShapeDtypeStruct((2, N, DH), jnp.float32),
    )(x, W1, degp)


def _tc2_call(acc1, hs1, degp, W2, b1, g1, bb1):
    return pl.pallas_call(
        _tc2_body,
        grid=(GRID,),
        in_specs=[_half_spec, _half_spec, _degp_spec, _mat_spec(D, D),
                  _vec_spec(D), _vec_spec(D), _vec_spec(D)],
        out_specs=[_row_spec(D), _half_spec],
        out_shape=[
            jax.ShapeDtypeStruct((N, D), jnp.float32),
            jax.ShapeDtypeStruct((2, N, DH), jnp.float32),
        ],
    )(acc1, hs1, degp, W2, b1, g1, bb1)


def _tc3a_call(x, h1, fc1_W, fc1_b):
    return pl.pallas_call(
        _tc3a_body,
        grid=(GRID,),
        in_specs=[_row_spec(D), _row_spec(D), _mat_spec(2 * D, D),
                  _vec_spec(D)],
        out_specs=_row_spec(D),
        out_shape=jax.ShapeDtypeStruct((N, D), jnp.float32),
    )(x, h1, fc1_W[:2 * D], fc1_b)


def _tc3b_call(u0, acc2, hs2, degp, b2, g2, bb2, fc1_W, fc2_W, fc2_b):
    return pl.pallas_call(
        _tc3b_body,
        grid=(GRID,),
        in_specs=[_row_spec(D), _half_spec, _half_spec,
                  _degp_spec, _vec_spec(D), _vec_spec(D), _vec_spec(D),
                  _mat_spec(D, D), _mat_spec(D, D), _vec_spec(D)],
        out_specs=_row_spec(D),
        out_shape=jax.ShapeDtypeStruct((N, D), jnp.float32),
    )(u0, acc2, hs2, degp, b2, g2, bb2, fc1_W[2 * D:], fc2_W, fc2_b)


def kernel(x, adj, weight, W1, b1, ln1_g, ln1_b, W2, b2, ln2_g, ln2_b,
           fc1_W, fc1_b, fc2_W, fc2_b):
    row = adj[0].astype(jnp.int32)
    col = adj[1].astype(jnp.int32)
    w = weight.astype(jnp.float32)

    b1r = b1.reshape(1, D)
    g1r = ln1_g.reshape(1, D)
    bb1r = ln1_b.reshape(1, D)
    b2r = b2.reshape(1, D)
    g2r = ln2_g.reshape(1, D)
    bb2r = ln2_b.reshape(1, D)
    fc1br = fc1_b.reshape(1, D)
    fc2br = fc2_b.reshape(1, D)

    degp = _deg_kernel(col, w).reshape(NTILES, N)
    row3 = row.reshape(NSUB, NCHUNK, CH)
    hs1 = _tc1_call(x, W1, degp)
    acc1 = _prop_kernel(hs1.reshape(2 * N, DH), row3, col, w)
    acc1 = acc1.reshape(2, N, DH)
    h1, hs2 = _tc2_call(acc1, hs1, degp, W2, b1r, g1r, bb1r)
    acc2 = _prop_kernel(hs2.reshape(2 * N, DH), row3, col, w)
    acc2 = acc2.reshape(2, N, DH)
    u0 = _tc3a_call(x, h1, fc1_W, fc1br)
    return _tc3b_call(u0, acc2, hs2, degp, b2r, g2r, bb2r,
                      fc1_W, fc2_W, fc2br)
